# baseline (speedup 1.0000x reference)
import jax, jax.numpy as jnp
from jax.experimental import pallas as pl


def kernel(x, edge_index, adj_tensor, Wl, Wr, b, Wa, ba, Wm, bm):
    h = x
    L = Wl.shape[0]
    n = x.shape[0]
    for l in range(L):
        def per_rel(ei, wl, wr, bb):
            src = ei[0]
            dst = ei[1]
            msg = h[src]
            agg = jax.ops.segment_sum(msg, dst, num_segments=n)
            cnt = jax.ops.segment_sum(jnp.ones((src.shape[0],), h.dtype), dst, num_segments=n)
            mean = agg / jnp.maximum(cnt, 1.0)[:, None]
            return mean @ wl + h @ wr + bb
        outs = jax.vmap(per_rel)(edge_index, Wl[l], Wr[l], b[l])
        h = jax.nn.relu(jnp.sum(outs, axis=0))
    add_coff = jax.nn.relu(h @ Wa + ba)
    mul_coff = jax.nn.relu(h @ Wm + bm)
    c = jnp.sum(mul_coff * add_coff, axis=-1)
    body = jnp.einsum('ijr,ir->ij', adj_tensor, mul_coff)
    return body + c[:, None]


# speedup vs baseline: 5.5077x; 5.5077x over previous
import functools
import numpy as np
import jax
import jax.numpy as jnp
from jax import lax
from jax.experimental import pallas as pl
from jax.experimental.pallas import tpu as pltpu
from jax.experimental.pallas import tpu_sc as plsc

N = 1024
R = 22
E = 16384
D = 128
NC = 2
NS = 16
RPC = R // NC
EPT = E // NS
CH = 128
NK = EPT // CH
ROWS_PER_TILE = RPC * N // NS


def _sc_agg_kernel():
    mesh = plsc.VectorSubcoreMesh(core_axis_name="c", subcore_axis_name="s")
    scratch = [
        pltpu.VMEM((CH,), jnp.int32),
        pltpu.VMEM((CH,), jnp.int32),
        pltpu.VMEM((CH, D), jnp.float32),
        pltpu.VMEM_SHARED((RPC * N, D), jnp.float32),
        pltpu.SemaphoreType.DMA,
    ]

    def body(h_hbm, srcp_hbm, dstp_hbm, zeros_hbm, agg_hbm,
             src_v, dst_v, rows_v, acc_sh, sem):
        c = lax.axis_index("c")
        s = lax.axis_index("s")

        pltpu.sync_copy(zeros_hbm.at[pl.ds(0, ROWS_PER_TILE)],
                        acc_sh.at[pl.ds(s * ROWS_PER_TILE, ROWS_PER_TILE)])
        plsc.subcore_barrier()

        for l in range(RPC):
            for k in range(NK):
                pltpu.sync_copy(srcp_hbm.at[c, l, s, k], src_v)
                pltpu.sync_copy(dstp_hbm.at[c, l, s, k], dst_v)
                pltpu.async_copy(h_hbm.at[src_v], rows_v, sem).wait()
                pltpu.sync_copy(rows_v, acc_sh.at[dst_v], add=True)
        plsc.subcore_barrier()

        for l in range(RPC):
            pltpu.sync_copy(
                acc_sh.at[pl.ds(l * N + s * (N // NS), N // NS)],
                agg_hbm.at[c, l, pl.ds(s * (N // NS), N // NS)])

    return pl.kernel(
        body,
        out_type=(jax.ShapeDtypeStruct((NC, RPC, N, D), jnp.float32),),
        mesh=mesh, scratch_types=scratch)


def _sc_cnt_kernel():
    mesh = plsc.VectorSubcoreMesh(core_axis_name="c", subcore_axis_name="s")
    scratch = [
        pltpu.VMEM((CH,), jnp.int32),
        pltpu.VMEM((CH, D), jnp.float32),
        pltpu.VMEM_SHARED((RPC * N, D), jnp.float32),
    ]

    def body(dstp_hbm, zeros_hbm, ones_hbm, cnt_hbm,
             dst_v, ones_v, cnt_sh):
        c = lax.axis_index("c")
        s = lax.axis_index("s")

        pltpu.sync_copy(ones_hbm, ones_v)
        pltpu.sync_copy(zeros_hbm.at[pl.ds(0, ROWS_PER_TILE)],
                        cnt_sh.at[pl.ds(s * ROWS_PER_TILE, ROWS_PER_TILE)])
        plsc.subcore_barrier()

        for l in range(RPC):
            for k in range(NK):
                pltpu.sync_copy(dstp_hbm.at[c, l, s, k], dst_v)
                pltpu.sync_copy(ones_v, cnt_sh.at[dst_v], add=True)
        plsc.subcore_barrier()

        for l in range(RPC):
            pltpu.sync_copy(
                cnt_sh.at[pl.ds(l * N + s * (N // NS), N // NS)],
                cnt_hbm.at[c, l, pl.ds(s * (N // NS), N // NS)])

    return pl.kernel(
        body,
        out_type=(jax.ShapeDtypeStruct((NC, RPC, N, D), jnp.float32),),
        mesh=mesh, scratch_types=scratch)


def _tc_layer_body(agg_ref, cnt_ref, h_ref, wlp_ref, wrs_ref, bs_ref,
                   out_ref):
    x = h_ref[...]
    acc = jnp.dot(x, wrs_ref[...], preferred_element_type=jnp.float32)
    acc += bs_ref[...]
    for r in range(R):
        cnt = jnp.maximum(cnt_ref[r, :, 0:1], 1.0)
        mean = agg_ref[r] / cnt
        acc += jnp.dot(mean, wlp_ref[r], preferred_element_type=jnp.float32)
    out_ref[...] = jnp.maximum(acc, 0.0)


BI = 64
GK = 22 * 128


def _tc_final_body(adj_ref, h_ref, wa_ref, ba_ref, wm_ref, bm_ref,
                   sel_ref, g2_ref, out_ref):
    hb = h_ref[...]
    A = jnp.maximum(jnp.dot(hb, wa_ref[...],
                            preferred_element_type=jnp.float32) + ba_ref[...],
                    0.0)
    M = jnp.maximum(jnp.dot(hb, wm_ref[...],
                            preferred_element_type=jnp.float32) + bm_ref[...],
                    0.0)
    cvec = jnp.sum(A * M, axis=1, keepdims=True)
    Pm = jnp.dot(M, sel_ref[...], preferred_element_type=jnp.float32)
    for g in range(N // 128):
        seg = adj_ref[:, g * GK:(g + 1) * GK] * Pm
        yg = jnp.dot(seg, g2_ref[...], preferred_element_type=jnp.float32)
        out_ref[:, g * 128:(g + 1) * 128] = yg + cvec


def kernel(x, edge_index, adj_tensor, Wl, Wr, b, Wa, ba, Wm, bm):
    f32 = jnp.float32
    perm = np.arange(R).reshape(RPC, NC).T.reshape(-1)
    src = edge_index[perm, 0, :]
    dst = edge_index[perm, 1, :]
    loff = (np.arange(R) % RPC)[:, None] * N
    dst = dst + loff.astype(np.int32)
    srcp = src.reshape(NC, RPC, NS, NK, CH)
    dstp = dst.reshape(NC, RPC, NS, NK, CH)

    zeros = jnp.zeros((N, D), f32)
    ones128 = jnp.ones((CH, D), f32)

    Wlp = Wl[:, perm]
    Wrs = jnp.sum(Wr, axis=1)
    bs = jnp.sum(b, axis=1)[:, None, :]
    pad = np.zeros((D, D - R), np.float32)
    Wa_p = jnp.concatenate([Wa, pad], axis=1)
    Wm_p = jnp.concatenate([Wm, pad], axis=1)
    ba_p = jnp.concatenate([ba, jnp.zeros((D - R,), f32)])[None, :]
    bm_p = jnp.concatenate([bm, jnp.zeros((D - R,), f32)])[None, :]

    k2 = np.arange(GK)
    sel = np.zeros((D, GK), np.float32)
    sel[k2 % R, k2] = 1.0
    g2 = np.zeros((GK, 128), np.float32)
    g2[k2, k2 // R] = 1.0
    sel = jnp.asarray(sel)
    g2 = jnp.asarray(g2)

    sc_agg = _sc_agg_kernel()
    sc_cnt = _sc_cnt_kernel()

    tc_layer = pl.pallas_call(
        _tc_layer_body,
        out_shape=jax.ShapeDtypeStruct((N, D), f32),
    )

    (cnt,) = sc_cnt(dstp, zeros, ones128)
    cnt = cnt.reshape(R, N, D)
    (agg1,) = sc_agg(x, srcp, dstp, zeros)
    agg1 = agg1.reshape(R, N, D)
    h1 = tc_layer(agg1, cnt, x, Wlp[0], Wrs[0], bs[0])

    (agg2,) = sc_agg(h1, srcp, dstp, zeros)
    agg2 = agg2.reshape(R, N, D)
    h2 = tc_layer(agg2, cnt, h1, Wlp[1], Wrs[1], bs[1])

    adj_flat = adj_tensor.reshape(N, N * R)
    grid = N // BI
    tc_final = pl.pallas_call(
        _tc_final_body,
        grid=(grid,),
        in_specs=[
            pl.BlockSpec((BI, N * R), lambda i: (i, 0)),
            pl.BlockSpec((BI, D), lambda i: (i, 0)),
            pl.BlockSpec((D, D), lambda i: (0, 0)),
            pl.BlockSpec((1, D), lambda i: (0, 0)),
            pl.BlockSpec((D, D), lambda i: (0, 0)),
            pl.BlockSpec((1, D), lambda i: (0, 0)),
            pl.BlockSpec((D, GK), lambda i: (0, 0)),
            pl.BlockSpec((GK, 128), lambda i: (0, 0)),
        ],
        out_specs=pl.BlockSpec((BI, N), lambda i: (i, 0)),
        out_shape=jax.ShapeDtypeStruct((N, N), f32),
    )
    return tc_final(adj_flat, h2, Wa_p, ba_p, Wm_p, bm_p, sel, g2)


# speedup vs baseline: 7.0525x; 1.2805x over previous
import functools
import numpy as np
import jax
import jax.numpy as jnp
from jax import lax
from jax.experimental import pallas as pl
from jax.experimental.pallas import tpu as pltpu
from jax.experimental.pallas import tpu_sc as plsc

N = 1024
R = 22
E = 16384
D = 128
NC = 2
NS = 16
RPC = R // NC
EPT = E // NS
CH = 128
NK = EPT // CH
ROWS_PER_TILE = RPC * N // NS


def _sc_agg_kernel():
    mesh = plsc.VectorSubcoreMesh(core_axis_name="c", subcore_axis_name="s")
    scratch = [
        pltpu.VMEM((2, CH), jnp.int32),
        pltpu.VMEM((2, CH), jnp.int32),
        pltpu.VMEM((CH, D), jnp.float32),
        pltpu.VMEM((CH, D), jnp.float32),
        pltpu.VMEM_SHARED((RPC * N, D), jnp.float32),
        pltpu.SemaphoreType.DMA,
        pltpu.SemaphoreType.DMA,
    ]

    def body(h_hbm, ei_hbm, zeros_hbm, agg_hbm,
             src_v, dst_v, rows0_v, rows1_v, acc_sh, sem0, sem1):
        c = lax.axis_index("c")
        s = lax.axis_index("s")
        rows = (rows0_v, rows1_v)
        sems = (sem0, sem1)
        chunks = [(l, k) for l in range(RPC) for k in range(NK)]

        def load_idx(j, buf):
            l, k = chunks[j]
            r = 2 * l + c
            pltpu.sync_copy(ei_hbm.at[r, 0, s, k], src_v.at[buf])
            pltpu.sync_copy(ei_hbm.at[r, 1, s, k], dst_v.at[buf])
            for i in range(CH // 16):
                dst_v[buf, pl.ds(i * 16, 16)] = (
                    dst_v[buf, pl.ds(i * 16, 16)] + l * N)

        pltpu.sync_copy(zeros_hbm.at[pl.ds(0, ROWS_PER_TILE)],
                        acc_sh.at[pl.ds(s * ROWS_PER_TILE, ROWS_PER_TILE)])
        plsc.subcore_barrier()

        load_idx(0, 0)
        pend = pltpu.async_copy(h_hbm.at[src_v.at[0]], rows[0], sems[0])
        for j in range(len(chunks)):
            cur = j % 2
            nxt = 1 - cur
            cur_pend = pend
            if j + 1 < len(chunks):
                load_idx(j + 1, nxt)
                pend = pltpu.async_copy(h_hbm.at[src_v.at[nxt]], rows[nxt],
                                        sems[nxt])
            cur_pend.wait()
            pltpu.sync_copy(rows[cur], acc_sh.at[dst_v.at[cur]], add=True)
        plsc.subcore_barrier()

        for l in range(RPC):
            pltpu.sync_copy(
                acc_sh.at[pl.ds(l * N + s * (N // NS), N // NS)],
                agg_hbm.at[c, l, pl.ds(s * (N // NS), N // NS)])

    return pl.kernel(
        body,
        out_type=(jax.ShapeDtypeStruct((NC, RPC, N, D), jnp.float32),),
        mesh=mesh, scratch_types=scratch)


def _sc_cnt_kernel():
    mesh = plsc.VectorSubcoreMesh(core_axis_name="c", subcore_axis_name="s")
    scratch = [
        pltpu.VMEM((CH,), jnp.int32),
        pltpu.VMEM((CH, D), jnp.float32),
        pltpu.VMEM_SHARED((RPC * N, D), jnp.float32),
    ]

    def body(ei_hbm, zeros_hbm, ones_hbm, cnt_hbm,
             dst_v, ones_v, cnt_sh):
        c = lax.axis_index("c")
        s = lax.axis_index("s")

        pltpu.sync_copy(ones_hbm, ones_v)
        pltpu.sync_copy(zeros_hbm.at[pl.ds(0, ROWS_PER_TILE)],
                        cnt_sh.at[pl.ds(s * ROWS_PER_TILE, ROWS_PER_TILE)])
        plsc.subcore_barrier()

        for l in range(RPC):
            for k in range(NK):
                pltpu.sync_copy(ei_hbm.at[2 * l + c, 1, s, k], dst_v)
                for i in range(CH // 16):
                    dst_v[pl.ds(i * 16, 16)] = (
                        dst_v[pl.ds(i * 16, 16)] + l * N)
                pltpu.sync_copy(ones_v, cnt_sh.at[dst_v], add=True)
        plsc.subcore_barrier()

        for l in range(RPC):
            pltpu.sync_copy(
                cnt_sh.at[pl.ds(l * N + s * (N // NS), N // NS)],
                cnt_hbm.at[c, l, pl.ds(s * (N // NS), N // NS)])

    return pl.kernel(
        body,
        out_type=(jax.ShapeDtypeStruct((NC, RPC, N, D), jnp.float32),),
        mesh=mesh, scratch_types=scratch)


def _tc_layer_body(agg_ref, cnt_ref, h_ref, wlp_ref, wrs_ref, bs_ref,
                   out_ref):
    x = h_ref[...]
    acc = jnp.dot(x, wrs_ref[...], preferred_element_type=jnp.float32)
    acc += bs_ref[...]
    for r in range(R):
        cnt = jnp.maximum(cnt_ref[r, :, 0:1], 1.0)
        mean = agg_ref[r] / cnt
        acc += jnp.dot(mean, wlp_ref[r], preferred_element_type=jnp.float32)
    out_ref[...] = jnp.maximum(acc, 0.0)


BI = 64
GK = 22 * 128


def _tc_final_body(adj_ref, h_ref, wa_ref, ba_ref, wm_ref, bm_ref,
                   sel_ref, g2_ref, out_ref):
    hb = h_ref[...]
    A = jnp.maximum(jnp.dot(hb, wa_ref[...],
                            preferred_element_type=jnp.float32) + ba_ref[...],
                    0.0)
    M = jnp.maximum(jnp.dot(hb, wm_ref[...],
                            preferred_element_type=jnp.float32) + bm_ref[...],
                    0.0)
    cvec = jnp.sum(A * M, axis=1, keepdims=True)
    Pm = jnp.dot(M, sel_ref[...], preferred_element_type=jnp.float32)
    for g in range(N // 128):
        seg = adj_ref[:, g * GK:(g + 1) * GK] * Pm
        yg = jnp.dot(seg, g2_ref[...], preferred_element_type=jnp.float32)
        out_ref[:, g * 128:(g + 1) * 128] = yg + cvec


def kernel(x, edge_index, adj_tensor, Wl, Wr, b, Wa, ba, Wm, bm):
    f32 = jnp.float32
    perm = np.arange(R).reshape(RPC, NC).T.reshape(-1)
    ei = edge_index.reshape(R, 2, NS, NK, CH)

    zeros = jnp.zeros((N, D), f32)
    ones128 = jnp.ones((CH, D), f32)

    Wlp = Wl[:, perm]
    Wrs = jnp.sum(Wr, axis=1)
    bs = jnp.sum(b, axis=1)[:, None, :]
    pad = np.zeros((D, D - R), np.float32)
    Wa_p = jnp.concatenate([Wa, pad], axis=1)
    Wm_p = jnp.concatenate([Wm, pad], axis=1)
    ba_p = jnp.concatenate([ba, jnp.zeros((D - R,), f32)])[None, :]
    bm_p = jnp.concatenate([bm, jnp.zeros((D - R,), f32)])[None, :]

    k2 = np.arange(GK)
    sel = np.zeros((D, GK), np.float32)
    sel[k2 % R, k2] = 1.0
    g2 = np.zeros((GK, 128), np.float32)
    g2[k2, k2 // R] = 1.0
    sel = jnp.asarray(sel)
    g2 = jnp.asarray(g2)

    sc_agg = _sc_agg_kernel()
    sc_cnt = _sc_cnt_kernel()

    tc_layer = pl.pallas_call(
        _tc_layer_body,
        out_shape=jax.ShapeDtypeStruct((N, D), f32),
    )

    (cnt,) = sc_cnt(ei, zeros, ones128)
    cnt = cnt.reshape(R, N, D)
    (agg1,) = sc_agg(x, ei, zeros)
    agg1 = agg1.reshape(R, N, D)
    h1 = tc_layer(agg1, cnt, x, Wlp[0], Wrs[0], bs[0])

    (agg2,) = sc_agg(h1, ei, zeros)
    agg2 = agg2.reshape(R, N, D)
    h2 = tc_layer(agg2, cnt, h1, Wlp[1], Wrs[1], bs[1])

    adj_flat = adj_tensor.reshape(N, N * R)
    grid = N // BI
    tc_final = pl.pallas_call(
        _tc_final_body,
        grid=(grid,),
        in_specs=[
            pl.BlockSpec((BI, N * R), lambda i: (i, 0)),
            pl.BlockSpec((BI, D), lambda i: (i, 0)),
            pl.BlockSpec((D, D), lambda i: (0, 0)),
            pl.BlockSpec((1, D), lambda i: (0, 0)),
            pl.BlockSpec((D, D), lambda i: (0, 0)),
            pl.BlockSpec((1, D), lambda i: (0, 0)),
            pl.BlockSpec((D, GK), lambda i: (0, 0)),
            pl.BlockSpec((GK, 128), lambda i: (0, 0)),
        ],
        out_specs=pl.BlockSpec((BI, N), lambda i: (i, 0)),
        out_shape=jax.ShapeDtypeStruct((N, N), f32),
    )
    return tc_final(adj_flat, h2, Wa_p, ba_p, Wm_p, bm_p, sel, g2)


# speedup vs baseline: 7.8200x; 1.1088x over previous
import functools
import numpy as np
import jax
import jax.numpy as jnp
from jax import lax
from jax.experimental import pallas as pl
from jax.experimental.pallas import tpu as pltpu
from jax.experimental.pallas import tpu_sc as plsc

N = 1024
R = 22
E = 16384
D = 128
NC = 2
NS = 16
RPC = R // NC
EPT = E // NS
CH = 128
NK = EPT // CH
ROWS_PER_TILE = RPC * N // NS


def _sc_agg_kernel():
    mesh = plsc.VectorSubcoreMesh(core_axis_name="c", subcore_axis_name="s")
    scratch = [
        pltpu.VMEM((2, CH), jnp.int32),
        pltpu.VMEM((2, CH), jnp.int32),
        pltpu.VMEM((CH, D), jnp.float32),
        pltpu.VMEM((CH, D), jnp.float32),
        pltpu.VMEM_SHARED((RPC * N, D), jnp.float32),
        pltpu.SemaphoreType.DMA,
        pltpu.SemaphoreType.DMA,
    ]

    def body(h_hbm, ei_hbm, zeros_hbm, agg_hbm,
             src_v, dst_v, rows0_v, rows1_v, acc_sh, sem0, sem1):
        c = lax.axis_index("c")
        s = lax.axis_index("s")
        rows = (rows0_v, rows1_v)
        sems = (sem0, sem1)
        chunks = [(l, k) for l in range(RPC) for k in range(NK)]

        def load_idx(j, buf):
            l, k = chunks[j]
            r = 2 * l + c
            pltpu.sync_copy(ei_hbm.at[r, 0, s, k], src_v.at[buf])
            pltpu.sync_copy(ei_hbm.at[r, 1, s, k], dst_v.at[buf])
            for i in range(CH // 16):
                dst_v[buf, pl.ds(i * 16, 16)] = (
                    dst_v[buf, pl.ds(i * 16, 16)] + l * N)

        pltpu.sync_copy(zeros_hbm.at[pl.ds(0, ROWS_PER_TILE)],
                        acc_sh.at[pl.ds(s * ROWS_PER_TILE, ROWS_PER_TILE)])
        plsc.subcore_barrier()

        load_idx(0, 0)
        pend = pltpu.async_copy(h_hbm.at[src_v.at[0]], rows[0], sems[0])
        for j in range(len(chunks)):
            cur = j % 2
            nxt = 1 - cur
            cur_pend = pend
            if j + 1 < len(chunks):
                load_idx(j + 1, nxt)
                pend = pltpu.async_copy(h_hbm.at[src_v.at[nxt]], rows[nxt],
                                        sems[nxt])
            cur_pend.wait()
            pltpu.sync_copy(rows[cur], acc_sh.at[dst_v.at[cur]], add=True)
        plsc.subcore_barrier()

        for l in range(RPC):
            pltpu.sync_copy(
                acc_sh.at[pl.ds(l * N + s * (N // NS), N // NS)],
                agg_hbm.at[c, l, pl.ds(s * (N // NS), N // NS)])

    return pl.kernel(
        body,
        out_type=(jax.ShapeDtypeStruct((NC, RPC, N, D), jnp.float32),),
        mesh=mesh, scratch_types=scratch)


def _sc_cnt_kernel():
    mesh = plsc.VectorSubcoreMesh(core_axis_name="c", subcore_axis_name="s")
    scratch = [
        pltpu.VMEM((CH,), jnp.int32),
        pltpu.VMEM((CH, D), jnp.float32),
        pltpu.VMEM_SHARED((RPC * N, D), jnp.float32),
    ]

    def body(ei_hbm, zeros_hbm, ones_hbm, cnt_hbm,
             dst_v, ones_v, cnt_sh):
        c = lax.axis_index("c")
        s = lax.axis_index("s")

        pltpu.sync_copy(ones_hbm, ones_v)
        pltpu.sync_copy(zeros_hbm.at[pl.ds(0, ROWS_PER_TILE)],
                        cnt_sh.at[pl.ds(s * ROWS_PER_TILE, ROWS_PER_TILE)])
        plsc.subcore_barrier()

        for l in range(RPC):
            for k in range(NK):
                pltpu.sync_copy(ei_hbm.at[2 * l + c, 1, s, k], dst_v)
                for i in range(CH // 16):
                    dst_v[pl.ds(i * 16, 16)] = (
                        dst_v[pl.ds(i * 16, 16)] + l * N)
                pltpu.sync_copy(ones_v, cnt_sh.at[dst_v], add=True)
        plsc.subcore_barrier()

        for l in range(RPC):
            pltpu.sync_copy(
                cnt_sh.at[pl.ds(l * N + s * (N // NS), N // NS)],
                cnt_hbm.at[c, l, pl.ds(s * (N // NS), N // NS)])

    return pl.kernel(
        body,
        out_type=(jax.ShapeDtypeStruct((NC, RPC, N, D), jnp.float32),),
        mesh=mesh, scratch_types=scratch)


def _tc_layer_body(agg_ref, cnt_ref, h_ref, wlp_ref, wrs_ref, bs_ref,
                   out_ref):
    x = h_ref[...]
    acc = jnp.dot(x, wrs_ref[...], preferred_element_type=jnp.float32)
    acc += bs_ref[...]
    for r in range(R):
        cnt = jnp.maximum(cnt_ref[r, :, 0:1], 1.0)
        mean = agg_ref[r] / cnt
        acc += jnp.dot(mean, wlp_ref[r], preferred_element_type=jnp.float32)
    out_ref[...] = jnp.maximum(acc, 0.0)


BI = 64


def _tc_final_body(adj_ref, h_ref, wa_ref, ba_ref, wm_ref, bm_ref, out_ref):
    hb = h_ref[...]
    A = jnp.maximum(jnp.dot(hb, wa_ref[...],
                            preferred_element_type=jnp.float32) + ba_ref[...],
                    0.0)
    M = jnp.maximum(jnp.dot(hb, wm_ref[...],
                            preferred_element_type=jnp.float32) + bm_ref[...],
                    0.0)
    cvec = jnp.sum(A * M, axis=1, keepdims=True)
    acc = jnp.broadcast_to(cvec, (BI, N))
    for r in range(R):
        acc = acc + adj_ref[:, r, :] * M[:, r:r + 1]
    out_ref[...] = acc


def kernel(x, edge_index, adj_tensor, Wl, Wr, b, Wa, ba, Wm, bm):
    f32 = jnp.float32
    perm = np.arange(R).reshape(RPC, NC).T.reshape(-1)
    ei = edge_index.reshape(R, 2, NS, NK, CH)

    zeros = jnp.zeros((N, D), f32)
    ones128 = jnp.ones((CH, D), f32)

    Wlp = Wl[:, perm]
    Wrs = jnp.sum(Wr, axis=1)
    bs = jnp.sum(b, axis=1)[:, None, :]
    pad = np.zeros((D, D - R), np.float32)
    Wa_p = jnp.concatenate([Wa, pad], axis=1)
    Wm_p = jnp.concatenate([Wm, pad], axis=1)
    ba_p = jnp.concatenate([ba, jnp.zeros((D - R,), f32)])[None, :]
    bm_p = jnp.concatenate([bm, jnp.zeros((D - R,), f32)])[None, :]

    sc_agg = _sc_agg_kernel()
    sc_cnt = _sc_cnt_kernel()

    tc_layer = pl.pallas_call(
        _tc_layer_body,
        out_shape=jax.ShapeDtypeStruct((N, D), f32),
    )

    (cnt,) = sc_cnt(ei, zeros, ones128)
    cnt = cnt.reshape(R, N, D)
    (agg1,) = sc_agg(x, ei, zeros)
    agg1 = agg1.reshape(R, N, D)
    h1 = tc_layer(agg1, cnt, x, Wlp[0], Wrs[0], bs[0])

    (agg2,) = sc_agg(h1, ei, zeros)
    agg2 = agg2.reshape(R, N, D)
    h2 = tc_layer(agg2, cnt, h1, Wlp[1], Wrs[1], bs[1])

    adj_t = jnp.swapaxes(adj_tensor, 1, 2)
    grid = N // BI
    tc_final = pl.pallas_call(
        _tc_final_body,
        grid=(grid,),
        in_specs=[
            pl.BlockSpec((BI, R, N), lambda i: (i, 0, 0)),
            pl.BlockSpec((BI, D), lambda i: (i, 0)),
            pl.BlockSpec((D, D), lambda i: (0, 0)),
            pl.BlockSpec((1, D), lambda i: (0, 0)),
            pl.BlockSpec((D, D), lambda i: (0, 0)),
            pl.BlockSpec((1, D), lambda i: (0, 0)),
        ],
        out_specs=pl.BlockSpec((BI, N), lambda i: (i, 0)),
        out_shape=jax.ShapeDtypeStruct((N, N), f32),
    )
    return tc_final(adj_t, h2, Wa_p, ba_p, Wm_p, bm_p)


# speedup vs baseline: 8.8296x; 1.1291x over previous
import functools
import numpy as np
import jax
import jax.numpy as jnp
from jax import lax
from jax.experimental import pallas as pl
from jax.experimental.pallas import tpu as pltpu
from jax.experimental.pallas import tpu_sc as plsc

N = 1024
R = 22
E = 16384
D = 128
NC = 2
NS = 16
RPC = R // NC
EPT = E // NS
CH = 128
NK = EPT // CH
ROWS_PER_TILE = RPC * N // NS


def _sc_agg_kernel():
    mesh = plsc.VectorSubcoreMesh(core_axis_name="c", subcore_axis_name="s")
    scratch = [
        pltpu.VMEM((2, CH), jnp.int32),
        pltpu.VMEM((2, CH), jnp.int32),
        pltpu.VMEM((CH, D), jnp.float32),
        pltpu.VMEM((CH, D), jnp.float32),
        pltpu.VMEM_SHARED((RPC * N, D), jnp.float32),
        pltpu.SemaphoreType.DMA,
        pltpu.SemaphoreType.DMA,
    ]

    def body(h_hbm, ei_hbm, zeros_hbm, agg_hbm,
             src_v, dst_v, rows0_v, rows1_v, acc_sh, sem0, sem1):
        c = lax.axis_index("c")
        s = lax.axis_index("s")
        rows = (rows0_v, rows1_v)
        sems = (sem0, sem1)
        chunks = [(l, k) for l in range(RPC) for k in range(NK)]

        def load_idx(j, buf):
            l, k = chunks[j]
            r = 2 * l + c
            pltpu.sync_copy(ei_hbm.at[r, 0, s, k], src_v.at[buf])
            pltpu.sync_copy(ei_hbm.at[r, 1, s, k], dst_v.at[buf])
            for i in range(CH // 16):
                dst_v[buf, pl.ds(i * 16, 16)] = (
                    dst_v[buf, pl.ds(i * 16, 16)] + l * N)

        pltpu.sync_copy(zeros_hbm.at[pl.ds(0, ROWS_PER_TILE)],
                        acc_sh.at[pl.ds(s * ROWS_PER_TILE, ROWS_PER_TILE)])
        plsc.subcore_barrier()

        load_idx(0, 0)
        pend = pltpu.async_copy(h_hbm.at[src_v.at[0]], rows[0], sems[0])
        for j in range(len(chunks)):
            cur = j % 2
            nxt = 1 - cur
            cur_pend = pend
            if j + 1 < len(chunks):
                load_idx(j + 1, nxt)
                pend = pltpu.async_copy(h_hbm.at[src_v.at[nxt]], rows[nxt],
                                        sems[nxt])
            cur_pend.wait()
            pltpu.sync_copy(rows[cur], acc_sh.at[dst_v.at[cur]], add=True)
        plsc.subcore_barrier()

        for l in range(RPC):
            pltpu.sync_copy(
                acc_sh.at[pl.ds(l * N + s * (N // NS), N // NS)],
                agg_hbm.at[c, l, pl.ds(s * (N // NS), N // NS)])

    return pl.kernel(
        body,
        out_type=(jax.ShapeDtypeStruct((NC, RPC, N, D), jnp.float32),),
        mesh=mesh, scratch_types=scratch)


def _sc_cnt_kernel():
    mesh = plsc.VectorSubcoreMesh(core_axis_name="c", subcore_axis_name="s")
    scratch = [
        pltpu.VMEM((CH,), jnp.int32),
        pltpu.VMEM((CH, D), jnp.float32),
        pltpu.VMEM_SHARED((RPC * N, D), jnp.float32),
    ]

    def body(ei_hbm, zeros_hbm, ones_hbm, cnt_hbm,
             dst_v, ones_v, cnt_sh):
        c = lax.axis_index("c")
        s = lax.axis_index("s")

        pltpu.sync_copy(ones_hbm, ones_v)
        pltpu.sync_copy(zeros_hbm.at[pl.ds(0, ROWS_PER_TILE)],
                        cnt_sh.at[pl.ds(s * ROWS_PER_TILE, ROWS_PER_TILE)])
        plsc.subcore_barrier()

        for l in range(RPC):
            for k in range(NK):
                pltpu.sync_copy(ei_hbm.at[2 * l + c, 1, s, k], dst_v)
                for i in range(CH // 16):
                    dst_v[pl.ds(i * 16, 16)] = (
                        dst_v[pl.ds(i * 16, 16)] + l * N)
                pltpu.sync_copy(ones_v, cnt_sh.at[dst_v], add=True)
        plsc.subcore_barrier()

        for l in range(RPC):
            pltpu.sync_copy(
                cnt_sh.at[pl.ds(l * N + s * (N // NS), N // NS)],
                cnt_hbm.at[c, l, pl.ds(s * (N // NS), N // NS)])

    return pl.kernel(
        body,
        out_type=(jax.ShapeDtypeStruct((NC, RPC, N, D), jnp.float32),),
        mesh=mesh, scratch_types=scratch)


def _tc_layer_body(agg_ref, cnt_ref, h_ref, wlp_ref, wrs_ref, bs_ref,
                   out_ref):
    x = h_ref[...]
    acc = jnp.dot(x, wrs_ref[...], preferred_element_type=jnp.float32)
    acc += bs_ref[...]
    for r in range(R):
        cnt = jnp.maximum(cnt_ref[r, :, 0:1], 1.0)
        mean = agg_ref[r] / cnt
        acc += jnp.dot(mean, wlp_ref[r], preferred_element_type=jnp.float32)
    out_ref[...] = jnp.maximum(acc, 0.0)


BI = 64


def _tc_final_body(adj_ref, h_ref, wa_ref, ba_ref, wm_ref, bm_ref, out_ref):
    hb = h_ref[...]
    A = jnp.maximum(jnp.dot(hb, wa_ref[...],
                            preferred_element_type=jnp.float32) + ba_ref[...],
                    0.0)
    M = jnp.maximum(jnp.dot(hb, wm_ref[...],
                            preferred_element_type=jnp.float32) + bm_ref[...],
                    0.0)
    cvec = jnp.sum(A * M, axis=1, keepdims=True)
    acc = jnp.broadcast_to(cvec, (BI, N))
    for r in range(R):
        acc = acc + adj_ref[r] * M[:, r:r + 1]
    out_ref[...] = acc


def kernel(x, edge_index, adj_tensor, Wl, Wr, b, Wa, ba, Wm, bm):
    f32 = jnp.float32
    perm = np.arange(R).reshape(RPC, NC).T.reshape(-1)
    ei = edge_index.reshape(R, 2, NS, NK, CH)

    zeros = jnp.zeros((N, D), f32)
    ones128 = jnp.ones((CH, D), f32)

    Wlp = Wl[:, perm]
    Wrs = jnp.sum(Wr, axis=1)
    bs = jnp.sum(b, axis=1)[:, None, :]
    pad = np.zeros((D, D - R), np.float32)
    Wa_p = jnp.concatenate([Wa, pad], axis=1)
    Wm_p = jnp.concatenate([Wm, pad], axis=1)
    ba_p = jnp.concatenate([ba, jnp.zeros((D - R,), f32)])[None, :]
    bm_p = jnp.concatenate([bm, jnp.zeros((D - R,), f32)])[None, :]

    sc_agg = _sc_agg_kernel()
    sc_cnt = _sc_cnt_kernel()

    tc_layer = pl.pallas_call(
        _tc_layer_body,
        out_shape=jax.ShapeDtypeStruct((N, D), f32),
    )

    (cnt,) = sc_cnt(ei, zeros, ones128)
    cnt = cnt.reshape(R, N, D)
    (agg1,) = sc_agg(x, ei, zeros)
    agg1 = agg1.reshape(R, N, D)
    h1 = tc_layer(agg1, cnt, x, Wlp[0], Wrs[0], bs[0])

    (agg2,) = sc_agg(h1, ei, zeros)
    agg2 = agg2.reshape(R, N, D)
    h2 = tc_layer(agg2, cnt, h1, Wlp[1], Wrs[1], bs[1])

    adj_t = jnp.transpose(adj_tensor, (2, 0, 1))
    grid = N // BI
    tc_final = pl.pallas_call(
        _tc_final_body,
        grid=(grid,),
        in_specs=[
            pl.BlockSpec((R, BI, N), lambda i: (0, i, 0)),
            pl.BlockSpec((BI, D), lambda i: (i, 0)),
            pl.BlockSpec((D, D), lambda i: (0, 0)),
            pl.BlockSpec((1, D), lambda i: (0, 0)),
            pl.BlockSpec((D, D), lambda i: (0, 0)),
            pl.BlockSpec((1, D), lambda i: (0, 0)),
        ],
        out_specs=pl.BlockSpec((BI, N), lambda i: (i, 0)),
        out_shape=jax.ShapeDtypeStruct((N, N), f32),
    )
    return tc_final(adj_t, h2, Wa_p, ba_p, Wm_p, bm_p)


# speedup vs baseline: 10.1510x; 1.1497x over previous
import functools
import numpy as np
import jax
import jax.numpy as jnp
from jax import lax
from jax.experimental import pallas as pl
from jax.experimental.pallas import tpu as pltpu
from jax.experimental.pallas import tpu_sc as plsc

N = 1024
R = 22
E = 16384
D = 128
NC = 2
NS = 16
RPC = R // NC
EPT = E // NS
CH = 128
NK = EPT // CH
ROWS_PER_TILE = RPC * N // NS


def _sc_agg_kernel():
    mesh = plsc.VectorSubcoreMesh(core_axis_name="c", subcore_axis_name="s")
    scratch = [
        pltpu.VMEM((2, NK, CH), jnp.int32),
        pltpu.VMEM((2, NK, CH), jnp.int32),
        pltpu.VMEM((CH, D), jnp.float32),
        pltpu.VMEM((CH, D), jnp.float32),
        pltpu.VMEM_SHARED((RPC * N, D), jnp.float32),
        pltpu.SemaphoreType.DMA,
        pltpu.SemaphoreType.DMA,
        pltpu.SemaphoreType.DMA,
    ]

    def body(h_hbm, ei_hbm, zeros_hbm, agg_hbm,
             src_v, dst_v, rows0_v, rows1_v, acc_sh, sem0, sem1, semi):
        c = lax.axis_index("c")
        s = lax.axis_index("s")
        rows = (rows0_v, rows1_v)
        sems = (sem0, sem1)

        def start_idx(l, buf):
            r = 2 * l + c
            cs = pltpu.async_copy(ei_hbm.at[r, 0, s], src_v.at[buf], semi)
            cd = pltpu.async_copy(ei_hbm.at[r, 1, s], dst_v.at[buf], semi)
            return (cs, cd)

        def finish_idx(l, buf, pend_idx):
            pend_idx[0].wait()
            pend_idx[1].wait()
            if l > 0:
                for k in range(NK):
                    for i in range(CH // 16):
                        dst_v[buf, k, pl.ds(i * 16, 16)] = (
                            dst_v[buf, k, pl.ds(i * 16, 16)] + l * N)

        pltpu.sync_copy(zeros_hbm.at[pl.ds(0, ROWS_PER_TILE)],
                        acc_sh.at[pl.ds(s * ROWS_PER_TILE, ROWS_PER_TILE)])
        plsc.subcore_barrier()

        finish_idx(0, 0, start_idx(0, 0))
        pend = pltpu.async_copy(h_hbm.at[src_v.at[0, 0]], rows[0], sems[0])
        pend_idx = None
        for l in range(RPC):
            ib = l % 2
            if l + 1 < RPC:
                pend_idx = start_idx(l + 1, 1 - ib)
            for k in range(NK):
                j = l * NK + k
                cur = j % 2
                nxt = 1 - cur
                cur_pend = pend
                if k + 1 < NK:
                    pend = pltpu.async_copy(
                        h_hbm.at[src_v.at[ib, k + 1]], rows[nxt], sems[nxt])
                elif l + 1 < RPC:
                    finish_idx(l + 1, 1 - ib, pend_idx)
                    pend = pltpu.async_copy(
                        h_hbm.at[src_v.at[1 - ib, 0]], rows[nxt], sems[nxt])
                cur_pend.wait()
                pltpu.sync_copy(rows[cur], acc_sh.at[dst_v.at[ib, k]],
                                add=True)
        plsc.subcore_barrier()

        for l in range(RPC):
            pltpu.sync_copy(
                acc_sh.at[pl.ds(l * N + s * (N // NS), N // NS)],
                agg_hbm.at[c, l, pl.ds(s * (N // NS), N // NS)])

    return pl.kernel(
        body,
        out_type=(jax.ShapeDtypeStruct((NC, RPC, N, D), jnp.float32),),
        mesh=mesh, scratch_types=scratch)


def _sc_cnt_kernel():
    mesh = plsc.VectorSubcoreMesh(core_axis_name="c", subcore_axis_name="s")
    scratch = [
        pltpu.VMEM((CH,), jnp.int32),
        pltpu.VMEM((CH, D), jnp.float32),
        pltpu.VMEM_SHARED((RPC * N, D), jnp.float32),
    ]

    def body(ei_hbm, zeros_hbm, ones_hbm, cnt_hbm,
             dst_v, ones_v, cnt_sh):
        c = lax.axis_index("c")
        s = lax.axis_index("s")

        pltpu.sync_copy(ones_hbm, ones_v)
        pltpu.sync_copy(zeros_hbm.at[pl.ds(0, ROWS_PER_TILE)],
                        cnt_sh.at[pl.ds(s * ROWS_PER_TILE, ROWS_PER_TILE)])
        plsc.subcore_barrier()

        for l in range(RPC):
            for k in range(NK):
                pltpu.sync_copy(ei_hbm.at[2 * l + c, 1, s, k], dst_v)
                for i in range(CH // 16):
                    dst_v[pl.ds(i * 16, 16)] = (
                        dst_v[pl.ds(i * 16, 16)] + l * N)
                pltpu.sync_copy(ones_v, cnt_sh.at[dst_v], add=True)
        plsc.subcore_barrier()

        for l in range(RPC):
            pltpu.sync_copy(
                cnt_sh.at[pl.ds(l * N + s * (N // NS), N // NS)],
                cnt_hbm.at[c, l, pl.ds(s * (N // NS), N // NS)])

    return pl.kernel(
        body,
        out_type=(jax.ShapeDtypeStruct((NC, RPC, N, D), jnp.float32),),
        mesh=mesh, scratch_types=scratch)


def _tc_layer_body(agg_ref, cnt_ref, h_ref, wlp_ref, wrs_ref, bs_ref,
                   out_ref):
    x = h_ref[...]
    acc = jnp.dot(x, wrs_ref[...], preferred_element_type=jnp.float32)
    acc += bs_ref[...]
    for r in range(R):
        cnt = jnp.maximum(cnt_ref[r, :, 0:1], 1.0)
        mean = agg_ref[r] / cnt
        acc += jnp.dot(mean, wlp_ref[r], preferred_element_type=jnp.float32)
    out_ref[...] = jnp.maximum(acc, 0.0)


BI = 64


def _tc_final_body(adj_ref, h_ref, wa_ref, ba_ref, wm_ref, bm_ref, out_ref):
    hb = h_ref[...]
    A = jnp.maximum(jnp.dot(hb, wa_ref[...],
                            preferred_element_type=jnp.float32) + ba_ref[...],
                    0.0)
    M = jnp.maximum(jnp.dot(hb, wm_ref[...],
                            preferred_element_type=jnp.float32) + bm_ref[...],
                    0.0)
    cvec = jnp.sum(A * M, axis=1, keepdims=True)
    acc = jnp.broadcast_to(cvec, (BI, N))
    for r in range(R):
        acc = acc + adj_ref[r] * M[:, r:r + 1]
    out_ref[...] = acc


def kernel(x, edge_index, adj_tensor, Wl, Wr, b, Wa, ba, Wm, bm):
    f32 = jnp.float32
    perm = np.arange(R).reshape(RPC, NC).T.reshape(-1)
    ei = edge_index.reshape(R, 2, NS, NK, CH)

    zeros = jnp.zeros((N, D), f32)
    ones128 = jnp.ones((CH, D), f32)

    Wlp = Wl[:, perm]
    Wrs = jnp.sum(Wr, axis=1)
    bs = jnp.sum(b, axis=1)[:, None, :]
    pad = np.zeros((D, D - R), np.float32)
    Wa_p = jnp.concatenate([Wa, pad], axis=1)
    Wm_p = jnp.concatenate([Wm, pad], axis=1)
    ba_p = jnp.concatenate([ba, jnp.zeros((D - R,), f32)])[None, :]
    bm_p = jnp.concatenate([bm, jnp.zeros((D - R,), f32)])[None, :]

    sc_agg = _sc_agg_kernel()
    sc_cnt = _sc_cnt_kernel()

    tc_layer = pl.pallas_call(
        _tc_layer_body,
        out_shape=jax.ShapeDtypeStruct((N, D), f32),
    )

    (cnt,) = sc_cnt(ei, zeros, ones128)
    cnt = cnt.reshape(R, N, D)
    (agg1,) = sc_agg(x, ei, zeros)
    agg1 = agg1.reshape(R, N, D)
    h1 = tc_layer(agg1, cnt, x, Wlp[0], Wrs[0], bs[0])

    (agg2,) = sc_agg(h1, ei, zeros)
    agg2 = agg2.reshape(R, N, D)
    h2 = tc_layer(agg2, cnt, h1, Wlp[1], Wrs[1], bs[1])

    adj_t = jnp.transpose(adj_tensor, (2, 0, 1))
    grid = N // BI
    tc_final = pl.pallas_call(
        _tc_final_body,
        grid=(grid,),
        in_specs=[
            pl.BlockSpec((R, BI, N), lambda i: (0, i, 0)),
            pl.BlockSpec((BI, D), lambda i: (i, 0)),
            pl.BlockSpec((D, D), lambda i: (0, 0)),
            pl.BlockSpec((1, D), lambda i: (0, 0)),
            pl.BlockSpec((D, D), lambda i: (0, 0)),
            pl.BlockSpec((1, D), lambda i: (0, 0)),
        ],
        out_specs=pl.BlockSpec((BI, N), lambda i: (i, 0)),
        out_shape=jax.ShapeDtypeStruct((N, N), f32),
    )
    return tc_final(adj_t, h2, Wa_p, ba_p, Wm_p, bm_p)


# speedup vs baseline: 10.1593x; 1.0008x over previous
import numpy as np
import jax
import jax.numpy as jnp
from jax import lax
from jax.experimental import pallas as pl
from jax.experimental.pallas import tpu as pltpu
from jax.experimental.pallas import tpu_sc as plsc

N = 1024
R = 22
E = 16384
D = 128
NC = 2
NS = 16
RPC = R // NC
EPT = E // NS
CH = 128
NK = EPT // CH
ROWS_PER_TILE = RPC * N // NS


def _sc_agg_kernel():
    mesh = plsc.VectorSubcoreMesh(core_axis_name="c", subcore_axis_name="s")
    scratch = [
        pltpu.VMEM((2, NK, CH), jnp.int32),
        pltpu.VMEM((2, NK, CH), jnp.int32),
        pltpu.VMEM((CH, D), jnp.float32),
        pltpu.VMEM((CH, D), jnp.float32),
        pltpu.VMEM_SHARED((RPC * N, D), jnp.float32),
        pltpu.SemaphoreType.DMA,
        pltpu.SemaphoreType.DMA,
        pltpu.SemaphoreType.DMA,
    ]

    def body(h_hbm, ei_hbm, zeros_hbm, agg_hbm,
             src_v, dst_v, rows0_v, rows1_v, acc_sh, sem0, sem1, semi):
        c = lax.axis_index("c")
        s = lax.axis_index("s")
        rows = (rows0_v, rows1_v)
        sems = (sem0, sem1)

        def start_idx(l, buf):
            r = 2 * l + c
            cs = pltpu.async_copy(ei_hbm.at[r, 0, s], src_v.at[buf], semi)
            cd = pltpu.async_copy(ei_hbm.at[r, 1, s], dst_v.at[buf], semi)
            return (cs, cd)

        def finish_idx(l, buf, pend_idx):
            pend_idx[0].wait()
            pend_idx[1].wait()
            if l > 0:
                for k in range(NK):
                    for i in range(CH // 16):
                        dst_v[buf, k, pl.ds(i * 16, 16)] = (
                            dst_v[buf, k, pl.ds(i * 16, 16)] + l * N)

        pltpu.sync_copy(zeros_hbm.at[pl.ds(0, ROWS_PER_TILE)],
                        acc_sh.at[pl.ds(s * ROWS_PER_TILE, ROWS_PER_TILE)])
        plsc.subcore_barrier()

        finish_idx(0, 0, start_idx(0, 0))
        pend = pltpu.async_copy(h_hbm.at[src_v.at[0, 0]], rows[0], sems[0])
        pend_idx = None
        for l in range(RPC):
            ib = l % 2
            if l + 1 < RPC:
                pend_idx = start_idx(l + 1, 1 - ib)
            for k in range(NK):
                j = l * NK + k
                cur = j % 2
                nxt = 1 - cur
                cur_pend = pend
                if k + 1 < NK:
                    pend = pltpu.async_copy(
                        h_hbm.at[src_v.at[ib, k + 1]], rows[nxt], sems[nxt])
                elif l + 1 < RPC:
                    finish_idx(l + 1, 1 - ib, pend_idx)
                    pend = pltpu.async_copy(
                        h_hbm.at[src_v.at[1 - ib, 0]], rows[nxt], sems[nxt])
                cur_pend.wait()
                pltpu.sync_copy(rows[cur], acc_sh.at[dst_v.at[ib, k]],
                                add=True)
        plsc.subcore_barrier()

        for l in range(RPC):
            pltpu.sync_copy(
                acc_sh.at[pl.ds(l * N + s * (N // NS), N // NS)],
                agg_hbm.at[c, l, pl.ds(s * (N // NS), N // NS)])

    return pl.kernel(
        body,
        out_type=(jax.ShapeDtypeStruct((NC, RPC, N, D), jnp.float32),),
        mesh=mesh, scratch_types=scratch)


def _sc_cnt_kernel():
    mesh = plsc.VectorSubcoreMesh(core_axis_name="c", subcore_axis_name="s")
    scratch = [
        pltpu.VMEM((CH,), jnp.int32),
        pltpu.VMEM((CH, D), jnp.float32),
        pltpu.VMEM_SHARED((RPC * N, D), jnp.float32),
    ]

    def body(ei_hbm, zeros_hbm, ones_hbm, cnt_hbm,
             dst_v, ones_v, cnt_sh):
        c = lax.axis_index("c")
        s = lax.axis_index("s")

        pltpu.sync_copy(ones_hbm, ones_v)
        pltpu.sync_copy(zeros_hbm.at[pl.ds(0, ROWS_PER_TILE)],
                        cnt_sh.at[pl.ds(s * ROWS_PER_TILE, ROWS_PER_TILE)])
        plsc.subcore_barrier()

        for l in range(RPC):
            for k in range(NK):
                pltpu.sync_copy(ei_hbm.at[2 * l + c, 1, s, k], dst_v)
                for i in range(CH // 16):
                    dst_v[pl.ds(i * 16, 16)] = (
                        dst_v[pl.ds(i * 16, 16)] + l * N)
                pltpu.sync_copy(ones_v, cnt_sh.at[dst_v], add=True)
        plsc.subcore_barrier()

        for l in range(RPC):
            pltpu.sync_copy(
                cnt_sh.at[pl.ds(l * N + s * (N // NS), N // NS)],
                cnt_hbm.at[c, l, pl.ds(s * (N // NS), N // NS)])

    return pl.kernel(
        body,
        out_type=(jax.ShapeDtypeStruct((NC, RPC, N, D), jnp.float32),),
        mesh=mesh, scratch_types=scratch)


def _tc_layer_body(agg_ref, cnt_ref, h_ref, wlp_ref, wrs_ref, bs_ref,
                   out_ref):
    x = h_ref[...]
    acc = jnp.dot(x, wrs_ref[...], preferred_element_type=jnp.float32)
    acc += bs_ref[...]
    for r in range(R):
        cnt = jnp.maximum(cnt_ref[r, :, 0:1], 1.0)
        mean = agg_ref[r] / cnt
        acc += jnp.dot(mean, wlp_ref[r], preferred_element_type=jnp.float32)
    out_ref[...] = jnp.maximum(acc, 0.0)


BI = 64


def _tc_final_body(adj_ref, h_ref, wa_ref, ba_ref, wm_ref, bm_ref, out_ref):
    hb = h_ref[...]
    A = jnp.maximum(jnp.dot(hb, wa_ref[...],
                            preferred_element_type=jnp.float32) + ba_ref[...],
                    0.0)
    M = jnp.maximum(jnp.dot(hb, wm_ref[...],
                            preferred_element_type=jnp.float32) + bm_ref[...],
                    0.0)
    cvec = jnp.sum(A * M, axis=1, keepdims=True)
    acc = jnp.broadcast_to(cvec, (BI, N))
    for r in range(R):
        acc = acc + adj_ref[r] * M[:, r:r + 1]
    out_ref[...] = acc


def kernel(x, edge_index, adj_tensor, Wl, Wr, b, Wa, ba, Wm, bm):
    f32 = jnp.float32
    perm = np.arange(R).reshape(RPC, NC).T.reshape(-1)
    ei = edge_index.reshape(R, 2, NS, NK, CH)

    zeros = jnp.zeros((N, D), f32)
    ones128 = jnp.ones((CH, D), f32)

    Wlp = Wl[:, perm]
    Wrs = jnp.sum(Wr, axis=1)
    bs = jnp.sum(b, axis=1)[:, None, :]
    pad = np.zeros((D, D - R), np.float32)
    Wa_p = jnp.concatenate([Wa, pad], axis=1)
    Wm_p = jnp.concatenate([Wm, pad], axis=1)
    ba_p = jnp.concatenate([ba, jnp.zeros((D - R,), f32)])[None, :]
    bm_p = jnp.concatenate([bm, jnp.zeros((D - R,), f32)])[None, :]

    sc_agg = _sc_agg_kernel()
    sc_cnt = _sc_cnt_kernel()

    tc_layer = pl.pallas_call(
        _tc_layer_body,
        out_shape=jax.ShapeDtypeStruct((N, D), f32),
    )

    (cnt,) = sc_cnt(ei, zeros, ones128)
    cnt = cnt.reshape(R, N, D)
    (agg1,) = sc_agg(x, ei, zeros)
    agg1 = agg1.reshape(R, N, D)
    h1 = tc_layer(agg1, cnt, x, Wlp[0], Wrs[0], bs[0])

    (agg2,) = sc_agg(h1, ei, zeros)
    agg2 = agg2.reshape(R, N, D)
    h2 = tc_layer(agg2, cnt, h1, Wlp[1], Wrs[1], bs[1])

    adj_t = jnp.transpose(adj_tensor, (2, 0, 1))
    grid = N // BI
    tc_final = pl.pallas_call(
        _tc_final_body,
        grid=(grid,),
        in_specs=[
            pl.BlockSpec((R, BI, N), lambda i: (0, i, 0)),
            pl.BlockSpec((BI, D), lambda i: (i, 0)),
            pl.BlockSpec((D, D), lambda i: (0, 0)),
            pl.BlockSpec((1, D), lambda i: (0, 0)),
            pl.BlockSpec((D, D), lambda i: (0, 0)),
            pl.BlockSpec((1, D), lambda i: (0, 0)),
        ],
        out_specs=pl.BlockSpec((BI, N), lambda i: (i, 0)),
        out_shape=jax.ShapeDtypeStruct((N, N), f32),
    )
    return tc_final(adj_t, h2, Wa_p, ba_p, Wm_p, bm_p)
